# baseline (speedup 1.0000x reference)
import jax
import jax.numpy as jnp
from jax.experimental import pallas as pl

_NUM_EMBEDDINGS = 1024
_EMBEDDING_DIM = 32
_COMMITMENT_COST = 0.25
_BATCHES_PER_STEP = 2


def _vq_kernel(lat_ref, emb_ref, embaug_ref, e2_ref, out_ref, loss_ref):
    b = pl.program_id(0)
    emb = emb_ref[...]
    embaug_t = embaug_ref[...]
    e2 = e2_ref[...]
    K = emb.shape[0]
    sq = jnp.zeros((1, 1), jnp.float32)
    for i in range(_BATCHES_PER_STEP):
        lat = lat_ref[i]
        hw = lat.shape[1]
        lat2 = lat * lat
        a = lat2[0:8] + lat2[8:16]
        a = a + lat2[16:24]
        a = a + lat2[24:32]
        a = a[0:4] + a[4:8]
        a = a[0:2] + a[2:4]
        f2 = a[0:1] + a[1:2]
        dt = (f2 + e2) - 2.0 * jnp.dot(emb, lat,
                                       preferred_element_type=jnp.float32)
        dmin = jnp.min(dt, axis=0, keepdims=True)
        m01 = dt == dmin
        onehot = m01.astype(jnp.float32)
        qa = jnp.dot(embaug_t, onehot,
                     preferred_element_type=jnp.float32)
        cnt = qa[-1:]
        qt = qa[:-1]

        def _tie_fix():
            iota = jax.lax.broadcasted_iota(jnp.int32, dt.shape, 0)
            idx = jnp.min(jnp.where(m01, iota, K), axis=0, keepdims=True)
            oh = (iota == idx).astype(jnp.float32)
            return jnp.dot(embaug_t, oh,
                           preferred_element_type=jnp.float32)[:-1]

        qt = jax.lax.cond(jnp.max(cnt) > 1.0, _tie_fix, lambda: qt)
        diff = lat - qt
        sq = sq + jnp.sum(diff * diff).reshape(1, 1)
        out_ref[i] = qt

    @pl.when(b == 0)
    def _init():
        loss_ref[...] = jnp.zeros((1, 1), jnp.float32)

    loss_ref[...] += sq


def kernel(latents, embedding):
    B, C, H, W = latents.shape
    K = embedding.shape[0]
    HW = H * W
    lat3 = latents.reshape(B, C, HW)
    e2_all = jnp.sum(embedding ** 2, axis=1).reshape(K, 1)
    emb_aug_t = jnp.concatenate(
        [embedding.T, jnp.ones((1, K), jnp.float32)], axis=0)

    nb = _BATCHES_PER_STEP
    out, loss_sum = pl.pallas_call(
        _vq_kernel,
        grid=(B // nb,),
        in_specs=[
            pl.BlockSpec((nb, C, HW), lambda b: (b, 0, 0)),
            pl.BlockSpec((K, C), lambda b: (0, 0)),
            pl.BlockSpec((C + 1, K), lambda b: (0, 0)),
            pl.BlockSpec((K, 1), lambda b: (0, 0)),
        ],
        out_specs=[
            pl.BlockSpec((nb, C, HW), lambda b: (b, 0, 0)),
            pl.BlockSpec((1, 1), lambda b: (0, 0)),
        ],
        out_shape=[
            jax.ShapeDtypeStruct((B, C, HW), jnp.float32),
            jax.ShapeDtypeStruct((1, 1), jnp.float32),
        ],
    )(lat3, embedding, emb_aug_t, e2_all)
    n_elems = B * C * HW
    vq_loss = (1.0 + _COMMITMENT_COST) * loss_sum[0, 0] / n_elems
    return out.reshape(B, C, H, W), vq_loss


# speedup vs baseline: 1.0025x; 1.0025x over previous
import jax
import jax.numpy as jnp
from jax.experimental import pallas as pl

_NUM_EMBEDDINGS = 1024
_EMBEDDING_DIM = 32
_COMMITMENT_COST = 0.25
_BATCHES_PER_STEP = 2


def _vq_kernel(lat_ref, emb_ref, embaug_ref, e2_ref, out_ref, loss_ref):
    b = pl.program_id(0)
    emb = emb_ref[...]
    embaug_t = embaug_ref[...]
    e2 = e2_ref[...]
    K = emb.shape[0]
    sq = jnp.zeros((1, 1), jnp.float32)
    for i in range(_BATCHES_PER_STEP):
        lat = lat_ref[i]
        hw = lat.shape[1]
        lat2 = lat * lat
        a = lat2[0:8] + lat2[8:16]
        a = a + lat2[16:24]
        a = a + lat2[24:32]
        a = a[0:4] + a[4:8]
        a = a[0:2] + a[2:4]
        f2 = a[0:1] + a[1:2]
        dt = (f2 + e2) - 2.0 * jnp.dot(emb, lat,
                                       preferred_element_type=jnp.float32)
        iota = jax.lax.broadcasted_iota(jnp.int32, dt.shape, 0)
        dmin = jnp.min(dt, axis=0, keepdims=True)
        idx = jnp.min(jnp.where(dt == dmin, iota, K),
                      axis=0, keepdims=True)
        onehot = (iota == idx).astype(jnp.float32)
        qt = jnp.dot(embaug_t[:-1], onehot,
                     preferred_element_type=jnp.float32)
        diff = lat - qt
        sq = sq + jnp.sum(diff * diff).reshape(1, 1)
        out_ref[i] = qt

    @pl.when(b == 0)
    def _init():
        loss_ref[...] = jnp.zeros((1, 1), jnp.float32)

    loss_ref[...] += sq


def kernel(latents, embedding):
    B, C, H, W = latents.shape
    K = embedding.shape[0]
    HW = H * W
    lat3 = latents.reshape(B, C, HW)
    e2_all = jnp.sum(embedding ** 2, axis=1).reshape(K, 1)
    emb_aug_t = jnp.concatenate(
        [embedding.T, jnp.ones((1, K), jnp.float32)], axis=0)

    nb = _BATCHES_PER_STEP
    out, loss_sum = pl.pallas_call(
        _vq_kernel,
        grid=(B // nb,),
        in_specs=[
            pl.BlockSpec((nb, C, HW), lambda b: (b, 0, 0)),
            pl.BlockSpec((K, C), lambda b: (0, 0)),
            pl.BlockSpec((C + 1, K), lambda b: (0, 0)),
            pl.BlockSpec((K, 1), lambda b: (0, 0)),
        ],
        out_specs=[
            pl.BlockSpec((nb, C, HW), lambda b: (b, 0, 0)),
            pl.BlockSpec((1, 1), lambda b: (0, 0)),
        ],
        out_shape=[
            jax.ShapeDtypeStruct((B, C, HW), jnp.float32),
            jax.ShapeDtypeStruct((1, 1), jnp.float32),
        ],
    )(lat3, embedding, emb_aug_t, e2_all)
    n_elems = B * C * HW
    vq_loss = (1.0 + _COMMITMENT_COST) * loss_sum[0, 0] / n_elems
    return out.reshape(B, C, H, W), vq_loss


# speedup vs baseline: 1.0455x; 1.0428x over previous
import jax
import jax.numpy as jnp
from jax.experimental import pallas as pl
from jax.experimental.pallas import tpu as pltpu

_NUM_EMBEDDINGS = 1024
_EMBEDDING_DIM = 32
_COMMITMENT_COST = 0.25
_BATCHES_PER_STEP = 2


def _vq_kernel(lat_ref, emb_ref, out_ref, loss_ref, e2_ref, embt_ref):
    b = pl.program_id(0)
    nsteps = pl.num_programs(0)
    emb = emb_ref[...]
    K = emb.shape[0]

    @pl.when(b == 0)
    def _prep():
        y2 = emb * emb
        g = y2[:, 0:8] + y2[:, 8:16]
        g = g + y2[:, 16:24]
        g = g + y2[:, 24:32]
        g = g[:, 0:4] + g[:, 4:8]
        g = g[:, 0:2] + g[:, 2:4]
        e2_ref[...] = g[:, 0:1] + g[:, 1:2]
        embt_ref[...] = emb.T
        loss_ref[...] = jnp.zeros((1, 1), jnp.float32)

    e2 = e2_ref[...]
    embt = embt_ref[...]
    sq = jnp.zeros((1, 1), jnp.float32)
    for i in range(_BATCHES_PER_STEP):
        lat = lat_ref[i]
        lat2 = lat * lat
        a = lat2[0:8] + lat2[8:16]
        a = a + lat2[16:24]
        a = a + lat2[24:32]
        a = a[0:4] + a[4:8]
        a = a[0:2] + a[2:4]
        f2 = a[0:1] + a[1:2]
        dt = (f2 + e2) - 2.0 * jnp.dot(emb, lat,
                                       preferred_element_type=jnp.float32)
        iota = jax.lax.broadcasted_iota(jnp.int32, dt.shape, 0)
        dmin = jnp.min(dt, axis=0, keepdims=True)
        idx = jnp.min(jnp.where(dt == dmin, iota, K),
                      axis=0, keepdims=True)
        onehot = (iota == idx).astype(jnp.float32)
        qt = jnp.dot(embt, onehot,
                     preferred_element_type=jnp.float32)
        diff = lat - qt
        sq = sq + jnp.sum(diff * diff).reshape(1, 1)
        out_ref[i] = qt

    loss_ref[...] += sq

    @pl.when(b == nsteps - 1)
    def _finish():
        n_elems = nsteps * _BATCHES_PER_STEP * lat_ref.shape[1] * lat_ref.shape[2]
        loss_ref[...] = loss_ref[...] * ((1.0 + _COMMITMENT_COST) / n_elems)


def kernel(latents, embedding):
    B, C, H, W = latents.shape
    K = embedding.shape[0]
    HW = H * W
    lat3 = latents.reshape(B, C, HW)

    nb = _BATCHES_PER_STEP
    out, loss = pl.pallas_call(
        _vq_kernel,
        grid=(B // nb,),
        in_specs=[
            pl.BlockSpec((nb, C, HW), lambda b: (b, 0, 0)),
            pl.BlockSpec((K, C), lambda b: (0, 0)),
        ],
        out_specs=[
            pl.BlockSpec((nb, C, HW), lambda b: (b, 0, 0)),
            pl.BlockSpec((1, 1), lambda b: (0, 0)),
        ],
        out_shape=[
            jax.ShapeDtypeStruct((B, C, HW), jnp.float32),
            jax.ShapeDtypeStruct((1, 1), jnp.float32),
        ],
        scratch_shapes=[
            pltpu.VMEM((K, 1), jnp.float32),
            pltpu.VMEM((C, K), jnp.float32),
        ],
    )(lat3, embedding)
    return out.reshape(B, C, H, W), loss[0, 0]
